# baseline (speedup 1.0000x reference)
import functools
import jax
import jax.numpy as jnp
from jax.experimental import pallas as pl
from jax.experimental.pallas import tpu as pltpu

N = 10000
E = 320000
D = 128
H = 8
C = 64
G = 64
OUT = 10

BN = 1000


def _mm1_body(x_ref, w_ref, ap_ref, h_ref, al_ref):
    xb = x_ref[...]
    hb = jnp.dot(xb, w_ref[...], preferred_element_type=jnp.float32)
    h_ref[...] = hb
    al_ref[...] = jnp.dot(hb, ap_ref[...], preferred_element_type=jnp.float32)


def _layer1_matmul(x, W1, Apack1):
    return pl.pallas_call(
        _mm1_body,
        grid=(N // BN,),
        in_specs=[
            pl.BlockSpec((BN, D), lambda i: (i, 0)),
            pl.BlockSpec((D, H * C), lambda i: (0, 0)),
            pl.BlockSpec((H * C, 2 * H), lambda i: (0, 0)),
        ],
        out_specs=[
            pl.BlockSpec((BN, H * C), lambda i: (i, 0)),
            pl.BlockSpec((BN, 2 * H), lambda i: (i, 0)),
        ],
        out_shape=[
            jax.ShapeDtypeStruct((N, H * C), jnp.float32),
            jax.ShapeDtypeStruct((N, 2 * H), jnp.float32),
        ],
    )(x, W1, Apack1)


def _mm2_body(acc_ref, den_ref, b_ref, w_ref, ap_ref, h2_ref, al_ref):
    acc = acc_ref[...].reshape(BN, H, C)
    den = den_ref[...][:, :, None]
    h1 = acc / (den + 1e-16) + b_ref[...].reshape(1, H, C)
    h1 = jnp.where(h1 > 0, h1, jnp.exp(jnp.minimum(h1, 0.0)) - 1.0).reshape(BN, H * C)
    h2 = jnp.dot(h1, w_ref[...], preferred_element_type=jnp.float32)
    h2_ref[...] = h2
    al_ref[...] = jnp.dot(h2, ap_ref[...], preferred_element_type=jnp.float32)


def _layer2_matmul(acc1, den1, b1, W2, Apack2):
    return pl.pallas_call(
        _mm2_body,
        grid=(N // BN,),
        in_specs=[
            pl.BlockSpec((BN, H * C), lambda i: (i, 0)),
            pl.BlockSpec((BN, H), lambda i: (i, 0)),
            pl.BlockSpec((1, H * C), lambda i: (0, 0)),
            pl.BlockSpec((H * C, C), lambda i: (0, 0)),
            pl.BlockSpec((C, 8), lambda i: (0, 0)),
        ],
        out_specs=[
            pl.BlockSpec((BN, C), lambda i: (i, 0)),
            pl.BlockSpec((BN, 8), lambda i: (i, 0)),
        ],
        out_shape=[
            jax.ShapeDtypeStruct((N, C), jnp.float32),
            jax.ShapeDtypeStruct((N, 8), jnp.float32),
        ],
    )(acc1, den1, b1.reshape(1, H * C), W2, Apack2)


def _final_body(acc_ref, den_ref, b_ref, batch_ref, wl_ref, bl_ref, out_ref):
    h2 = acc_ref[...] / (den_ref[...][:, 0:1] + 1e-16) + b_ref[...]
    gids = jax.lax.broadcasted_iota(jnp.int32, (G, N), 0)
    onehot = jnp.where(batch_ref[...] == gids, 1.0, 0.0)
    sums = jnp.dot(onehot, h2, preferred_element_type=jnp.float32)
    counts = jnp.sum(onehot, axis=1, keepdims=True)
    pooled = sums / jnp.maximum(counts, 1.0)
    logits = jnp.dot(pooled, wl_ref[...], preferred_element_type=jnp.float32)
    logits = logits + bl_ref[...]
    m = jnp.max(logits, axis=1, keepdims=True)
    lse = m + jnp.log(jnp.sum(jnp.exp(logits - m), axis=1, keepdims=True))
    out_ref[...] = logits - lse


def _final(acc2, den2, b2, batch2d, Wl, bl):
    return pl.pallas_call(
        _final_body,
        out_shape=jax.ShapeDtypeStruct((G, OUT), jnp.float32),
    )(acc2, den2, b2.reshape(1, C), batch2d, Wl, bl.reshape(1, OUT))


def _edge_pass_jnp(alphas, h, src, dst, heads, outc):
    a_s = alphas[src, :heads]
    a_d = alphas[dst, heads:2 * heads]
    e = a_s + a_d
    e = jnp.where(e >= 0, e, 0.2 * e)
    gmax = jax.lax.stop_gradient(jnp.max(e))
    ee = jnp.exp(e - gmax)
    den = jax.ops.segment_sum(ee, dst, num_segments=N)
    hr = h[src].reshape(-1, heads, outc)
    acc = jax.ops.segment_sum(hr * ee[:, :, None], dst, num_segments=N)
    acc = acc.reshape(-1, heads * outc)
    if heads < 8:
        den = jnp.pad(den, ((0, 0), (0, 8 - heads)))
    return acc, den


def kernel(x, edge_index, batch, W1, a_src1, a_dst1, b1, W2, a_src2, a_dst2, b2, Wl, bl):
    loop = jnp.arange(N, dtype=edge_index.dtype)
    src = jnp.concatenate([edge_index[0], loop])
    dst = jnp.concatenate([edge_index[1], loop])

    hc = H * C
    Apack1 = jnp.zeros((hc, 2 * H), jnp.float32)
    rows = jnp.arange(hc)
    Apack1 = Apack1.at[rows, rows // C].set(a_src1.reshape(-1))
    Apack1 = Apack1.at[rows, H + rows // C].set(a_dst1.reshape(-1))
    Apack2 = jnp.zeros((C, 8), jnp.float32)
    Apack2 = Apack2.at[jnp.arange(C), 0].set(a_src2.reshape(-1))
    Apack2 = Apack2.at[jnp.arange(C), 1].set(a_dst2.reshape(-1))

    h1, alphas1 = _layer1_matmul(x, W1, Apack1)
    acc1, den1 = _edge_pass_jnp(alphas1, h1, src, dst, H, C)
    h2, alphas2 = _layer2_matmul(acc1, den1, b1, W2, Apack2)
    acc2, den2 = _edge_pass_jnp(alphas2, h2, src, dst, 1, C)
    return _final(acc2, den2, b2, batch.reshape(1, N), Wl, bl)
